# baseline (speedup 1.0000x reference)
import dataclasses

import jax
import jax.numpy as jnp
from jax import lax
from jax.experimental import pallas as pl
from jax.experimental.pallas import tpu as pltpu
from jax.experimental.pallas import tpu_sc as plsc

VOCAB_ROWS = 1000000
BATCH = 16384
FIELDS = 26
N_EMBED = 64
NUM_IDX = BATCH * FIELDS

NUM_CORES = 2
NUM_SUBCORES = 16
NUM_WORKERS = NUM_CORES * NUM_SUBCORES

K_BATCH = 8
W_ROWS = K_BATCH * FIELDS
WINDOWS_PER_WORKER = BATCH // (K_BATCH * NUM_WORKERS)

C_ROWS = 200
NCHUNKS = VOCAB_ROWS // C_ROWS
PAD_ITERS = -(-NCHUNKS // NUM_WORKERS)
PAD_ITERS += PAD_ITERS % 2


def _compiler_params():
    cp = pltpu.CompilerParams()
    if "needs_layout_passes" in pltpu.CompilerParams.__dataclass_fields__:
        cp = dataclasses.replace(cp, needs_layout_passes=False)
    return cp


def _mesh():
    return plsc.VectorSubcoreMesh(core_axis_name="core", subcore_axis_name="subcore")


def _sc_pad(table):
    @pl.kernel(
        out_type=jax.ShapeDtypeStruct((VOCAB_ROWS, 2 * N_EMBED), jnp.float32),
        mesh=_mesh(),
        scratch_types=[
            pltpu.VMEM((C_ROWS, N_EMBED), jnp.float32),
            pltpu.VMEM((C_ROWS, N_EMBED), jnp.float32),
            pltpu.VMEM((C_ROWS, 2 * N_EMBED), jnp.float32),
            pltpu.VMEM((C_ROWS, 2 * N_EMBED), jnp.float32),
            pltpu.SemaphoreType.DMA,
            pltpu.SemaphoreType.DMA,
            pltpu.SemaphoreType.DMA,
            pltpu.SemaphoreType.DMA,
        ],
    )
    def kern(tab_hbm, scr_hbm, t0, t1, s0, s1, si0, si1, so0, so1):
        wid = lax.axis_index("core") * NUM_SUBCORES + lax.axis_index("subcore")

        bufs = ((t0, s0, si0, so0), (t1, s1, si1, so1))

        def chunk_of(t):
            return t * NUM_WORKERS + wid

        def start_in(t, t64, si):
            c = chunk_of(t)
            @pl.when(c < NCHUNKS)
            def _():
                pltpu.async_copy(tab_hbm.at[pl.ds(c * C_ROWS, C_ROWS)], t64, si)

        for b in range(2):
            start_in(b, bufs[b][0], bufs[b][2])

        def do_chunk(t, t64, s128, si, so):
            c = chunk_of(t)
            @pl.when(jnp.logical_and(t >= 2, chunk_of(t - 2) < NCHUNKS))
            def _():
                pltpu.make_async_copy(s128, scr_hbm.at[pl.ds(0, C_ROWS)], so).wait()

            @pl.when(c < NCHUNKS)
            def _():
                pltpu.make_async_copy(
                    tab_hbm.at[pl.ds(c * C_ROWS, C_ROWS)], t64, si
                ).wait()
                s128[:, 0:N_EMBED] = t64[...]
                start_in(t + 2, t64, si)
                pltpu.async_copy(s128, scr_hbm.at[pl.ds(c * C_ROWS, C_ROWS)], so)

        @pl.loop(0, PAD_ITERS, step=2)
        def _(t):
            for b in range(2):
                t64, s128, si, so = bufs[b]
                do_chunk(t + b, t64, s128, si, so)

        for b in range(2):
            t64, s128, si, so = bufs[b]
            @pl.when(chunk_of(PAD_ITERS - 2 + b) < NCHUNKS)
            def _():
                pltpu.make_async_copy(s128, scr_hbm.at[pl.ds(0, C_ROWS)], so).wait()

    return kern(table)


def _sc_gather(scr, idx_windows):
    @pl.kernel(
        out_type=jax.ShapeDtypeStruct((BATCH, FIELDS, N_EMBED), jnp.float32),
        mesh=_mesh(),
        scratch_types=[
            pltpu.VMEM((W_ROWS,), jnp.int32),
            pltpu.VMEM((W_ROWS,), jnp.int32),
            pltpu.VMEM((W_ROWS, 2 * N_EMBED), jnp.float32),
            pltpu.VMEM((W_ROWS, 2 * N_EMBED), jnp.float32),
            pltpu.VMEM((W_ROWS, N_EMBED), jnp.float32),
            pltpu.VMEM((W_ROWS, N_EMBED), jnp.float32),
            pltpu.SemaphoreType.DMA,
            pltpu.SemaphoreType.DMA,
            pltpu.SemaphoreType.DMA,
            pltpu.SemaphoreType.DMA,
        ],
    )
    def kern(scr_hbm, idx_hbm, out_hbm,
             idx_v0, idx_v1, g_v0, g_v1, o_v0, o_v1,
             gsem0, gsem1, osem0, osem1):
        wid = lax.axis_index("core") * NUM_SUBCORES + lax.axis_index("subcore")
        w_base = wid * WINDOWS_PER_WORKER

        bufs = ((idx_v0, g_v0, o_v0, gsem0, osem0),
                (idx_v1, g_v1, o_v1, gsem1, osem1))

        def start_gather(w, idx_v, g_v, gsem):
            pltpu.sync_copy(idx_hbm.at[w_base + w], idx_v)
            pltpu.async_copy(scr_hbm.at[idx_v], g_v, gsem)

        for b in range(2):
            idx_v, g_v, _, gsem, _ = bufs[b]
            start_gather(b, idx_v, g_v, gsem)

        def do_window(w, idx_v, g_v, o_v, gsem, osem):
            b0 = (w_base + w) * K_BATCH
            pltpu.make_async_copy(scr_hbm.at[idx_v], g_v, gsem).wait()

            @pl.when(w >= 2)
            def _():
                for j in range(K_BATCH):
                    pltpu.make_async_copy(
                        o_v.at[pl.ds(j * FIELDS, FIELDS)], out_hbm.at[0], osem
                    ).wait()

            o_v[...] = g_v[:, 0:N_EMBED]

            @pl.when(w + 2 < WINDOWS_PER_WORKER)
            def _():
                start_gather(w + 2, idx_v, g_v, gsem)

            for j in range(K_BATCH):
                pltpu.async_copy(
                    o_v.at[pl.ds(j * FIELDS, FIELDS)], out_hbm.at[b0 + j], osem
                )

        @pl.loop(0, WINDOWS_PER_WORKER, step=2)
        def _(w):
            for b in range(2):
                idx_v, g_v, o_v, gsem, osem = bufs[b]
                do_window(w + b, idx_v, g_v, o_v, gsem, osem)

        for b in range(2):
            _, _, o_v, _, osem = bufs[b]
            for j in range(K_BATCH):
                pltpu.make_async_copy(
                    o_v.at[pl.ds(j * FIELDS, FIELDS)], out_hbm.at[0], osem
                ).wait()

    return kern(scr, idx_windows)


def kernel(discrete, embeddings):
    idx_windows = discrete.astype(jnp.int32).reshape(NUM_IDX // W_ROWS, W_ROWS)
    scr = _sc_pad(embeddings)
    return _sc_gather(scr, idx_windows)


# speedup vs baseline: 1.3439x; 1.3439x over previous
import jax
import jax.numpy as jnp
from jax import lax
from jax.experimental import pallas as pl
from jax.experimental.pallas import tpu as pltpu
from jax.experimental.pallas import tpu_sc as plsc

VOCAB_ROWS = 1000000
BATCH = 16384
FIELDS = 26
N_EMBED = 64
NUM_IDX = BATCH * FIELDS

NUM_CORES = 2
NUM_SUBCORES = 16
NUM_WORKERS = NUM_CORES * NUM_SUBCORES

K_BATCH = 8
W_ROWS = K_BATCH * FIELDS
WINDOWS_PER_WORKER = BATCH // (K_BATCH * NUM_WORKERS)


def _sc_gather(scr, idx_windows):
    mesh = plsc.VectorSubcoreMesh(core_axis_name="core", subcore_axis_name="subcore")

    @pl.kernel(
        out_type=jax.ShapeDtypeStruct((BATCH, FIELDS, N_EMBED), jnp.float32),
        mesh=mesh,
        scratch_types=[
            pltpu.VMEM((W_ROWS,), jnp.int32),
            pltpu.VMEM((W_ROWS,), jnp.int32),
            pltpu.VMEM((W_ROWS, 2 * N_EMBED), jnp.float32),
            pltpu.VMEM((W_ROWS, 2 * N_EMBED), jnp.float32),
            pltpu.VMEM((W_ROWS, N_EMBED), jnp.float32),
            pltpu.VMEM((W_ROWS, N_EMBED), jnp.float32),
            pltpu.SemaphoreType.DMA,
            pltpu.SemaphoreType.DMA,
            pltpu.SemaphoreType.DMA,
            pltpu.SemaphoreType.DMA,
        ],
    )
    def kern(scr_hbm, idx_hbm, out_hbm,
             idx_v0, idx_v1, g_v0, g_v1, o_v0, o_v1,
             gsem0, gsem1, osem0, osem1):
        wid = lax.axis_index("core") * NUM_SUBCORES + lax.axis_index("subcore")
        w_base = wid * WINDOWS_PER_WORKER

        bufs = ((idx_v0, g_v0, o_v0, gsem0, osem0),
                (idx_v1, g_v1, o_v1, gsem1, osem1))

        def start_gather(w, idx_v, g_v, gsem):
            pltpu.sync_copy(idx_hbm.at[w_base + w], idx_v)
            pltpu.async_copy(scr_hbm.at[idx_v], g_v, gsem)

        for b in range(2):
            idx_v, g_v, _, gsem, _ = bufs[b]
            start_gather(b, idx_v, g_v, gsem)

        def do_window(w, idx_v, g_v, o_v, gsem, osem):
            b0 = (w_base + w) * K_BATCH
            pltpu.make_async_copy(scr_hbm.at[idx_v], g_v, gsem).wait()
            @pl.when(w >= 2)
            def _():
                for j in range(K_BATCH):
                    pltpu.make_async_copy(
                        o_v.at[pl.ds(j * FIELDS, FIELDS)], out_hbm.at[0], osem
                    ).wait()

            o_v[...] = g_v[:, 0:N_EMBED]

            @pl.when(w + 2 < WINDOWS_PER_WORKER)
            def _():
                start_gather(w + 2, idx_v, g_v, gsem)

            for j in range(K_BATCH):
                pltpu.async_copy(
                    o_v.at[pl.ds(j * FIELDS, FIELDS)], out_hbm.at[b0 + j], osem
                )

        @pl.loop(0, WINDOWS_PER_WORKER, step=2)
        def _(w):
            for b in range(2):
                idx_v, g_v, o_v, gsem, osem = bufs[b]
                do_window(w + b, idx_v, g_v, o_v, gsem, osem)

        for b in range(2):
            _, _, o_v, _, osem = bufs[b]
            for j in range(K_BATCH):
                pltpu.make_async_copy(
                    o_v.at[pl.ds(j * FIELDS, FIELDS)], out_hbm.at[0], osem
                ).wait()

    return kern(scr, idx_windows)


def kernel(discrete, embeddings):
    idx_windows = discrete.astype(jnp.int32).reshape(NUM_IDX // W_ROWS, W_ROWS)
    scr = jnp.pad(embeddings, ((0, 0), (0, N_EMBED)))
    return _sc_gather(scr, idx_windows)
